# baseline (speedup 1.0000x reference)
import functools

import jax
import jax.numpy as jnp
from jax.experimental import pallas as pl


def _logits_body(x_ref, w_ref, out_ref):
    xb = x_ref[0]
    w = w_ref[...]
    out_ref[0] = jax.lax.dot_general(
        w, xb, (((1,), (1,)), ((), ())),
        preferred_element_type=jnp.float32)


def _rank_scatter_body(lrow_ref, lcol_ref, w_ref, i_ref, *, it, t):
    step = pl.program_id(1)
    lrow = lrow_ref[0]
    lcol = lcol_ref[0]
    jj = jax.lax.broadcasted_iota(jnp.int32, (it, t), 1)
    ii = jax.lax.broadcasted_iota(jnp.int32, (it, t), 0) + step * it
    gt = lrow > lcol
    tie = (lrow == lcol) & (jj < ii)
    rank = jnp.sum((gt | tie).astype(jnp.int32), axis=1, keepdims=True)
    onehot = rank == jj
    sig = jax.nn.sigmoid(lcol)
    iv = (jax.lax.broadcasted_iota(jnp.int32, (it, 1), 0)
          + step * it).astype(jnp.float32)
    wpart = jnp.sum(jnp.where(onehot, sig, 0.0), axis=0, keepdims=True)
    ipart = jnp.sum(jnp.where(onehot, iv, 0.0), axis=0, keepdims=True)

    @pl.when(step == 0)
    def _():
        w_ref[...] = jnp.zeros_like(w_ref)
        i_ref[...] = jnp.zeros_like(i_ref)

    w_ref[0] += wpart
    i_ref[0] += ipart


def kernel(x, W, capacity_factor):
    B, T, C = x.shape
    TT = min(512, T)
    IT = min(512, T)

    logits = pl.pallas_call(
        _logits_body,
        grid=(B, T // TT),
        in_specs=[
            pl.BlockSpec((1, TT, C), lambda b, t: (b, t, 0)),
            pl.BlockSpec((1, C), lambda b, t: (0, 0)),
        ],
        out_specs=pl.BlockSpec((1, 1, TT), lambda b, t: (b, 0, t)),
        out_shape=jax.ShapeDtypeStruct((B, 1, T), jnp.float32),
    )(x, W)

    lrow = logits
    lcol = logits.reshape(B, T, 1)

    wsort, isort = pl.pallas_call(
        functools.partial(_rank_scatter_body, it=IT, t=T),
        grid=(B, T // IT),
        in_specs=[
            pl.BlockSpec((1, 1, T), lambda b, i: (b, 0, 0)),
            pl.BlockSpec((1, IT, 1), lambda b, i: (b, i, 0)),
        ],
        out_specs=[
            pl.BlockSpec((1, 1, T), lambda b, i: (b, 0, 0)),
            pl.BlockSpec((1, 1, T), lambda b, i: (b, 0, 0)),
        ],
        out_shape=[
            jax.ShapeDtypeStruct((B, 1, T), jnp.float32),
            jax.ShapeDtypeStruct((B, 1, T), jnp.float32),
        ],
    )(lrow, lcol)

    weights = wsort.reshape(B, T, 1)
    selected_tokens = isort.astype(jnp.int32).reshape(B, T, 1)
    is_final = jnp.zeros((B, T), dtype=bool)
    return (is_final, selected_tokens, weights)


# speedup vs baseline: 1.5057x; 1.5057x over previous
import functools

import jax
import jax.numpy as jnp
from jax import lax
from jax.experimental import pallas as pl
from jax.experimental.pallas import tpu as pltpu
from jax.experimental.pallas import tpu_sc as plsc

_L = 16
_NBIN = 256
_UNROLL = 4


def _logits_body(x_ref, w_ref, out_ref):
    xb = x_ref[0]
    w = w_ref[...]
    out_ref[0] = jax.lax.dot_general(
        w, xb, (((1,), (1,)), ((), ())),
        preferred_element_type=jnp.float32)


def _sc_sort_body(nbatch, t, logit_hbm, sig_hbm, idx_hbm,
                  lg, ka, kb, pa, pb, hist, tot, dbase, sg):
    lane = lax.iota(jnp.int32, _L)
    per_lane = t // _L
    lane_blk = lane * per_lane
    lane_hist = lane * _NBIN
    zero16 = jnp.zeros((_L,), jnp.int32)
    ones16 = jnp.ones((_L,), jnp.int32)
    nvec = t // _L

    wid = lax.axis_index("s") * 2 + lax.axis_index("c")

    @pl.when(wid < nbatch)
    def _():
        row = wid * t
        pltpu.sync_copy(logit_hbm.at[pl.ds(row, t)], lg)

        def mk(jo, _):
            for u in range(_UNROLL):
                seq = (jo * _UNROLL + u) * _L + lane
                v = plsc.bitcast(plsc.load_gather(lg, [seq]), jnp.int32)
                m = lax.shift_right_arithmetic(v, 31)
                k = v ^ ((m ^ jnp.int32(-1)) & jnp.int32(0x7FFFFFFF))
                plsc.store_scatter(ka, [seq], k)
                plsc.store_scatter(pa, [seq], seq)
            return 0
        lax.fori_loop(0, nvec // _UNROLL, mk, 0)

        def radix_pass(src_k, src_p, dst_k, dst_p, shift):
            def z(jo, _):
                for u in range(_UNROLL):
                    plsc.store_scatter(
                        hist, [(jo * _UNROLL + u) * _L + lane], zero16)
                return 0
            lax.fori_loop(0, (_L * _NBIN) // (_L * _UNROLL), z, 0)

            def h(jo, _):
                for u in range(_UNROLL):
                    j = jo * _UNROLL + u
                    k = plsc.load_gather(src_k, [lane_blk + j])
                    d = lax.shift_right_logical(k, shift) & 255
                    plsc.addupdate_scatter(hist, [lane_hist + d], ones16)
                return 0
            lax.fori_loop(0, nvec // _UNROLL, h, 0)

            def b1(c, _):
                dchunk = c * _L + lane
                s = zero16
                for l in range(_L):
                    hl = plsc.load_gather(hist, [l * _NBIN + dchunk])
                    plsc.store_scatter(hist, [l * _NBIN + dchunk], s)
                    s = s + hl
                plsc.store_scatter(tot, [dchunk], s)
                return 0
            lax.fori_loop(0, _NBIN // _L, b1, 0)

            def b2(c, carry):
                v = plsc.load_gather(tot, [c * _L + lane])
                incl = plsc.cumsum(v)
                plsc.store_scatter(dbase, [c * _L + lane],
                                   (incl - v) + carry)
                return carry + jnp.sum(v)
            lax.fori_loop(0, _NBIN // _L, b2, jnp.int32(0))

            def b3(c, _):
                dchunk = c * _L + lane
                bv = plsc.load_gather(dbase, [dchunk])
                for l in range(_L):
                    p = plsc.load_gather(hist, [l * _NBIN + dchunk])
                    plsc.store_scatter(hist, [l * _NBIN + dchunk], p + bv)
                return 0
            lax.fori_loop(0, _NBIN // _L, b3, 0)

            def sc(jo, _):
                for u in range(_UNROLL):
                    j = jo * _UNROLL + u
                    k = plsc.load_gather(src_k, [lane_blk + j])
                    p = plsc.load_gather(src_p, [lane_blk + j])
                    d = lax.shift_right_logical(k, shift) & 255
                    hidx = lane_hist + d
                    dest = plsc.load_gather(hist, [hidx])
                    plsc.addupdate_scatter(hist, [hidx], ones16)
                    plsc.store_scatter(dst_k, [dest], k)
                    plsc.store_scatter(dst_p, [dest], p)
                return 0
            lax.fori_loop(0, nvec // _UNROLL, sc, 0)

        radix_pass(ka, pa, kb, pb, 0)
        radix_pass(kb, pb, ka, pa, 8)
        radix_pass(ka, pa, kb, pb, 16)
        radix_pass(kb, pb, ka, pa, 24)

        def fin(jo, _):
            for u in range(_UNROLL):
                seq = (jo * _UNROLL + u) * _L + lane
                pidx = plsc.load_gather(pa, [seq])
                v = plsc.load_gather(lg, [pidx])
                s = 1.0 / (1.0 + jnp.exp(-v))
                plsc.store_scatter(sg, [seq], s)
            return 0
        lax.fori_loop(0, nvec // _UNROLL, fin, 0)

        pltpu.sync_copy(sg, sig_hbm.at[pl.ds(row, t)])
        pltpu.sync_copy(pa, idx_hbm.at[pl.ds(row, t)])


def kernel(x, W, capacity_factor):
    B, T, C = x.shape
    TT = min(1024, T)

    logits = pl.pallas_call(
        _logits_body,
        grid=(B, T // TT),
        in_specs=[
            pl.BlockSpec((1, TT, C), lambda b, t: (b, t, 0)),
            pl.BlockSpec((1, C), lambda b, t: (0, 0)),
        ],
        out_specs=pl.BlockSpec((1, 1, TT), lambda b, t: (b, 0, t)),
        out_shape=jax.ShapeDtypeStruct((B, 1, T), jnp.float32),
    )(x, W)

    mesh = plsc.VectorSubcoreMesh(core_axis_name="c", subcore_axis_name="s")
    sc_sort = functools.partial(
        pl.kernel,
        mesh=mesh,
        out_type=[
            jax.ShapeDtypeStruct((B * T,), jnp.float32),
            jax.ShapeDtypeStruct((B * T,), jnp.int32),
        ],
        scratch_types=[
            pltpu.VMEM((T,), jnp.float32),
            pltpu.VMEM((T,), jnp.int32),
            pltpu.VMEM((T,), jnp.int32),
            pltpu.VMEM((T,), jnp.int32),
            pltpu.VMEM((T,), jnp.int32),
            pltpu.VMEM((_L * _NBIN,), jnp.int32),
            pltpu.VMEM((_NBIN,), jnp.int32),
            pltpu.VMEM((_NBIN,), jnp.int32),
            pltpu.VMEM((T,), jnp.float32),
        ],
        compiler_params=pltpu.CompilerParams(
            use_tc_tiling_on_sc=False, needs_layout_passes=False),
    )(functools.partial(_sc_sort_body, B, T))

    sig_flat, idx_flat = sc_sort(logits.reshape(B * T))

    weights = sig_flat.reshape(B, T, 1)
    selected_tokens = idx_flat.reshape(B, T, 1)
    is_final = jnp.zeros((B, T), dtype=bool)
    return (is_final, selected_tokens, weights)
